# baseline (speedup 1.0000x reference)
import functools

import jax
import jax.numpy as jnp
from jax import lax
from jax.experimental import pallas as pl
from jax.experimental.pallas import tpu as pltpu
from jax.experimental.pallas import tpu_sc as plsc

B, S, T, P, E, GH = 64, 64, 24, 24, 512, 32
N_NODE = B * S * T
N_EDGE = T * B * E
N_GRAPH = T * B

_NC, _NS = 2, 16
_NW = _NC * _NS
_EPW = N_EDGE // _NW
_GPW = N_GRAPH // _NW
_XBASE = _GPW * T
_XSL = 1200
_CH = 1024
_NCHUNK = _EPW // _CH
_RPS = N_NODE // _NS


def _sc_segment_acc(x_flat, z4, rowl3, col3, e03, e13):
    mesh = plsc.VectorSubcoreMesh(core_axis_name="c", subcore_axis_name="s")
    nbuf = 2

    @functools.partial(
        pl.kernel,
        mesh=mesh,
        out_type=jax.ShapeDtypeStruct((_NC, 4, N_NODE), jnp.float32),
        scratch_types=[
            pltpu.VMEM((_XSL,), jnp.float32),
            [pltpu.VMEM((_CH,), jnp.int32) for _ in range(nbuf)],
            [pltpu.VMEM((8, 128), jnp.int32) for _ in range(nbuf)],
            [pltpu.VMEM((_CH,), jnp.float32) for _ in range(nbuf)],
            [pltpu.VMEM((_CH,), jnp.float32) for _ in range(nbuf)],
            [pltpu.VMEM((_CH,), jnp.float32) for _ in range(nbuf)],
            pltpu.VMEM((128,), jnp.float32),
            [pltpu.VMEM_SHARED((N_NODE,), jnp.float32) for _ in range(4)],
            pltpu.SemaphoreType.DMA,
            pltpu.SemaphoreType.DMA,
        ],
        compiler_params=pltpu.CompilerParams(needs_layout_passes=False),
    )
    def k(x_hbm, z_hbm, rowl_hbm, col_hbm, e0_hbm, e1_hbm, out_hbm,
          xv, rls, cols, e0s, e1s, xrs, onesv, accs, sem_in, sem_sc):
        cid = lax.axis_index("c")
        sid = lax.axis_index("s")
        wid = sid * _NC + cid

        for z, acc in enumerate(accs):
            pltpu.sync_copy(z_hbm, acc.at[pl.ds(sid * _RPS, _RPS)])

        def fill1(i, _):
            onesv[pl.ds(i * 16, 16)] = jnp.ones((16,), jnp.float32)
            return 0
        lax.fori_loop(0, 8, fill1, 0)
        plsc.subcore_barrier()

        pltpu.sync_copy(x_hbm.at[pl.ds(wid * _XBASE, _XSL)], xv)

        def start_in(ch, kb):
            cg = wid * _NCHUNK + ch
            return [
                pltpu.async_copy(rowl_hbm.at[cg], rls[kb], sem_in),
                pltpu.async_copy(col_hbm.at[cg], cols[kb], sem_in),
                pltpu.async_copy(e0_hbm.at[cg], e0s[kb], sem_in),
                pltpu.async_copy(e1_hbm.at[cg], e1s[kb], sem_in),
            ]

        pend_in = {0: start_in(0, 0)}
        pend_sc = {}
        for ch in range(_NCHUNK):
            kb = ch % nbuf
            for d in pend_in.pop(ch):
                d.wait()
            rl, e0v, e1v, xrv = rls[kb], e0s[kb], e1s[kb], xrs[kb]

            def build(j, _):
                base = j * 16
                r = rl[pl.ds(base, 16)]
                xrv[pl.ds(base, 16)] = plsc.load_gather(xv, [r])
                return 0
            lax.fori_loop(0, _CH // 16, build, 0)

            descs = []
            for r8 in range(8):
                idx = cols[kb].at[r8]
                sl = pl.ds(r8 * 128, 128)
                descs.append(pltpu.async_copy(
                    xrv.at[sl], accs[0].at[idx], sem_sc, add=True))
                descs.append(pltpu.async_copy(
                    onesv, accs[1].at[idx], sem_sc, add=True))
                descs.append(pltpu.async_copy(
                    e0v.at[sl], accs[2].at[idx], sem_sc, add=True))
                descs.append(pltpu.async_copy(
                    e1v.at[sl], accs[3].at[idx], sem_sc, add=True))
            pend_sc[ch] = descs

            for d in pend_sc.pop(ch - 1, []):
                d.wait()
            if ch + 1 < _NCHUNK:
                pend_in[ch + 1] = start_in(ch + 1, (ch + 1) % nbuf)

        for chx in sorted(pend_sc):
            for d in pend_sc[chx]:
                d.wait()
        plsc.subcore_barrier()
        sl = pl.ds(sid * _RPS, _RPS)
        for z, acc in enumerate(accs):
            pltpu.sync_copy(acc.at[sl], out_hbm.at[cid, z, sl])

    return k(x_flat, z4, rowl3, col3, e03, e13)


_BB = 16
_F32 = jnp.float32


def _tc_body(xT_ref, alT_ref, accT_ref, suT_ref, sf_ref, h0T_ref, c0T_ref,
             w1T_ref, w2T_ref, b1c_ref, b2c_ref,
             wieT_ref, wheT_ref, benT_ref,
             widT_ref, whdT_ref, bdnT_ref,
             wlinT_ref, blin_ref, out_ref, uproj_s, gxs):
    blk = pl.program_id(0)
    dot = functools.partial(jnp.dot, preferred_element_type=_F32)

    @pl.when(blk == 0)
    def _():
        w2uT = w2T_ref[:, 1 + GH:]
        up = dot(w2uT, suT_ref[...]) + jnp.broadcast_to(b2c_ref[...],
                                                        (GH, N_GRAPH))
        for p in range(T):
            uproj_s[p] = up[:, p * S:(p + 1) * S]

    w2mT = w2T_ref[:, 1:1 + GH]
    aT = dot(w2mT, w1T_ref[...])
    abT = dot(w2mT, b1c_ref[...])
    r6T = jnp.concatenate([w2T_ref[:, 0:1], aT, abT], axis=1)

    for j in range(_BB):
        sx = accT_ref[0, j, 0:1, :] + accT_ref[1, j, 0:1, :]
        cnt = accT_ref[0, j, 1:2, :] + accT_ref[1, j, 1:2, :]
        se0 = accT_ref[0, j, 2:3, :] + accT_ref[1, j, 2:3, :]
        se1 = accT_ref[0, j, 3:4, :] + accT_ref[1, j, 3:4, :]
        rc = 1.0 / jnp.maximum(cnt, 1.0)
        xj = xT_ref[j]
        cts = jnp.concatenate(
            [xj, sx * rc, xj * cnt * rc, se0 * rc, se1 * rc, cnt * rc],
            axis=0)
        for p in range(T):
            gxs[j, p] = (dot(r6T, cts[:, p * S:(p + 1) * S]) + uproj_s[p])

    wieT = wieT_ref[...]
    wheT = wheT_ref[...]
    benb = jnp.broadcast_to(benT_ref[...], (4 * GH, S))

    def sig(x):
        return 0.5 * jnp.tanh(0.5 * x) + 0.5

    def step(h, c, g):
        i = sig(g[0:GH])
        f = sig(g[GH:2 * GH])
        gg = jnp.tanh(g[2 * GH:3 * GH])
        o = sig(g[3 * GH:4 * GH])
        c = f * c + i * gg
        h = o * jnp.tanh(c)
        return h, c

    widT = widT_ref[...]
    whdT = whdT_ref[...]
    wlinT = wlinT_ref[...]
    blv = blin_ref[0, 0]
    wd0T = widT[:, 0:1]
    widrT = widT[:, 1:]
    whd2T = whdT + dot(wd0T, wlinT)
    colmask = (lax.broadcasted_iota(jnp.int32, (1, P), 1) >= 1).astype(_F32)
    dbias = (jnp.broadcast_to(bdnT_ref[...], (4 * GH, P))
             + dot(wd0T * blv, colmask))
    ones1s = jnp.ones((1, S), _F32)

    grp = 4
    for g0 in range(0, _BB, grp):
        js = list(range(g0, g0 + grp))

        def enc_body(p, carry):
            nxt = []
            for q, j in enumerate(js):
                h, c = carry[2 * q], carry[2 * q + 1]
                g = dot(wieT, gxs[j, p]) + dot(wheT, h) + benb
                h, c = step(h, c, g)
                nxt += [h, c]
            return tuple(nxt)

        init = []
        for j in js:
            init += [h0T_ref[j], c0T_ref[j]]
        st = lax.fori_loop(0, T, enc_body, tuple(init))
        hs = {j: st[2 * q] for q, j in enumerate(js)}
        cs = {j: st[2 * q + 1] for q, j in enumerate(js)}

        fpbs = {j: dot(widrT, sf_ref[j]) + dbias for j in js}
        for i in range(P):
            for j in js:
                if i == 0:
                    g = (dot(whdT, hs[j]) + dot(wd0T, alT_ref[j])
                         + dot(fpbs[j][:, 0:1], ones1s))
                else:
                    g = dot(whd2T, hs[j]) + dot(fpbs[j][:, i:i + 1], ones1s)
                hs[j], cs[j] = step(hs[j], cs[j], g)
                out_ref[j, i:i + 1, :] = dot(wlinT, hs[j]) + blv


def _tc_forward(xT, alT, accT, suT, sf, h0T, c0T,
                w1T, w2T, b1c, b2c, wieT, wheT, benT,
                widT, whdT, bdnT, wlinT, blin):
    grid = (B // _BB,)
    full = lambda shape: pl.BlockSpec(shape, lambda b: tuple(0 for _ in shape))
    specs = [
        pl.BlockSpec((_BB, 1, S * T), lambda b: (b, 0, 0)),
        pl.BlockSpec((_BB, 1, S), lambda b: (b, 0, 0)),
        pl.BlockSpec((2, _BB, 4, S * T), lambda b: (0, b, 0, 0)),
        full((43, N_GRAPH)),
        pl.BlockSpec((_BB, 11, P), lambda b: (b, 0, 0)),
        pl.BlockSpec((_BB, GH, S), lambda b: (b, 0, 0)),
        pl.BlockSpec((_BB, GH, S), lambda b: (b, 0, 0)),
        full((GH, 4)), full((GH, 1 + GH + 43)), full((GH, 1)), full((GH, 1)),
        full((4 * GH, GH)), full((4 * GH, GH)), full((4 * GH, 1)),
        full((4 * GH, 12)), full((4 * GH, GH)), full((4 * GH, 1)),
        full((1, GH)), full((1, 1)),
    ]
    return pl.pallas_call(
        _tc_body,
        grid=grid,
        in_specs=specs,
        out_specs=pl.BlockSpec((_BB, P, S), lambda b: (b, 0, 0)),
        out_shape=jax.ShapeDtypeStruct((B, P, S), jnp.float32),
        scratch_shapes=[
            pltpu.VMEM((T, GH, S), jnp.float32),
            pltpu.VMEM((_BB, T, GH, S), jnp.float32),
        ],
        compiler_params=pltpu.CompilerParams(
            dimension_semantics=("arbitrary",),
        ),
    )(xT, alT, accT, suT, sf, h0T, c0T,
      w1T, w2T, b1c, b2c, wieT, wheT, benT,
      widT, whdT, bdnT, wlinT, blin)


def kernel(sta_aqi, sta_conn, sta_w, sta_wea, sta_for, city_u,
           W1, b1, W2, b2, Wi_en, Wh_en, bi_en, bh_en,
           Wi_de, Wh_de, bi_de, bh_de, Wlin, blin, h0, c0):
    conn_t = jnp.transpose(sta_conn, (0, 2, 1))
    conn_g = jnp.tile(conn_t, (T, 1, 1))
    off = (jnp.arange(N_GRAPH, dtype=jnp.int32) * T)[:, None]
    row = (conn_g[:, 0, :] + off).reshape(-1)
    col = (conn_g[:, 1, :] + off).reshape(-1)
    widx = jnp.arange(N_EDGE, dtype=jnp.int32) // _EPW
    rowl = row - widx * _XBASE
    ew = sta_w.reshape(-1, 2)
    x_flat = sta_aqi.reshape(-1)

    rowl3 = rowl.reshape(N_EDGE // _CH, _CH)
    col3 = col.reshape(N_EDGE // 128, 128).reshape(N_EDGE // _CH, 8, 128)
    e03 = ew[:, 0].reshape(N_EDGE // _CH, _CH)
    e13 = ew[:, 1].reshape(N_EDGE // _CH, _CH)

    z4 = jnp.zeros((_RPS,), jnp.float32)
    acc = _sc_segment_acc(x_flat, z4, rowl3, col3, e03, e13)
    accT = jnp.transpose(acc.reshape(2, 4, B, S * T), (0, 2, 1, 3))

    xT = x_flat.reshape(B, 1, S * T)
    alT = sta_aqi[:, :, -1, 0].reshape(B, 1, S)
    suT = jnp.transpose(
        jnp.concatenate([city_u, sta_wea], axis=-1).reshape(N_GRAPH, 43))
    sfT = jnp.transpose(sta_for, (0, 2, 1))
    h0T = jnp.transpose(h0.reshape(B, S, GH), (0, 2, 1))
    c0T = jnp.transpose(c0.reshape(B, S, GH), (0, 2, 1))

    out = _tc_forward(
        xT, alT, accT, suT, sfT, h0T, c0T,
        jnp.transpose(W1), jnp.transpose(W2),
        b1.reshape(GH, 1), b2.reshape(GH, 1),
        jnp.transpose(Wi_en), jnp.transpose(Wh_en),
        (bi_en + bh_en).reshape(4 * GH, 1),
        jnp.transpose(Wi_de), jnp.transpose(Wh_de),
        (bi_de + bh_de).reshape(4 * GH, 1),
        jnp.transpose(Wlin), blin.reshape(1, 1))
    return jnp.transpose(out, (0, 2, 1))


# speedup vs baseline: 1.0322x; 1.0322x over previous
import functools

import jax
import jax.numpy as jnp
from jax import lax
from jax.experimental import pallas as pl
from jax.experimental.pallas import tpu as pltpu
from jax.experimental.pallas import tpu_sc as plsc

B, S, T, P, E, GH = 64, 64, 24, 24, 512, 32
N_NODE = B * S * T
N_EDGE = T * B * E
N_GRAPH = T * B

_NC, _NS = 2, 16
_NW = _NC * _NS
_EPW = N_EDGE // _NW
_GPW = N_GRAPH // _NW
_XBASE = _GPW * T
_XSL = 1200
_CH = 1024
_NCHUNK = _EPW // _CH
_RPS = N_NODE // _NS


def _sc_segment_acc(x_flat, z4, rowl3, col3, e03, e13):
    mesh = plsc.VectorSubcoreMesh(core_axis_name="c", subcore_axis_name="s")
    nbuf = 2

    @functools.partial(
        pl.kernel,
        mesh=mesh,
        out_type=jax.ShapeDtypeStruct((_NC, 4, N_NODE), jnp.float32),
        scratch_types=[
            pltpu.VMEM((_XSL,), jnp.float32),
            [pltpu.VMEM((_CH,), jnp.int32) for _ in range(nbuf)],
            [pltpu.VMEM((8, 128), jnp.int32) for _ in range(nbuf)],
            [pltpu.VMEM((_CH,), jnp.float32) for _ in range(nbuf)],
            [pltpu.VMEM((_CH,), jnp.float32) for _ in range(nbuf)],
            [pltpu.VMEM((_CH,), jnp.float32) for _ in range(nbuf)],
            pltpu.VMEM((128,), jnp.float32),
            [pltpu.VMEM_SHARED((N_NODE,), jnp.float32) for _ in range(4)],
            pltpu.SemaphoreType.DMA,
            pltpu.SemaphoreType.DMA,
        ],
        compiler_params=pltpu.CompilerParams(needs_layout_passes=False),
    )
    def k(x_hbm, z_hbm, rowl_hbm, col_hbm, e0_hbm, e1_hbm, out_hbm,
          xv, rls, cols, e0s, e1s, xrs, onesv, accs, sem_in, sem_sc):
        cid = lax.axis_index("c")
        sid = lax.axis_index("s")
        wid = sid * _NC + cid

        for z, acc in enumerate(accs):
            pltpu.sync_copy(z_hbm, acc.at[pl.ds(sid * _RPS, _RPS)])

        def fill1(i, _):
            onesv[pl.ds(i * 16, 16)] = jnp.ones((16,), jnp.float32)
            return 0
        lax.fori_loop(0, 8, fill1, 0)
        plsc.subcore_barrier()

        pltpu.sync_copy(x_hbm.at[pl.ds(wid * _XBASE, _XSL)], xv)

        def start_in(ch, kb):
            cg = wid * _NCHUNK + ch
            return [
                pltpu.async_copy(rowl_hbm.at[cg], rls[kb], sem_in),
                pltpu.async_copy(col_hbm.at[cg], cols[kb], sem_in),
                pltpu.async_copy(e0_hbm.at[cg], e0s[kb], sem_in),
                pltpu.async_copy(e1_hbm.at[cg], e1s[kb], sem_in),
            ]

        pend_in = {0: start_in(0, 0)}
        pend_sc = {}
        for ch in range(_NCHUNK):
            kb = ch % nbuf
            for d in pend_in.pop(ch):
                d.wait()
            rl, e0v, e1v, xrv = rls[kb], e0s[kb], e1s[kb], xrs[kb]

            def build(j, _):
                base = j * 16
                r = rl[pl.ds(base, 16)]
                xrv[pl.ds(base, 16)] = plsc.load_gather(xv, [r])
                return 0
            lax.fori_loop(0, _CH // 16, build, 0)

            descs = []
            for r8 in range(8):
                idx = cols[kb].at[r8]
                sl = pl.ds(r8 * 128, 128)
                descs.append(pltpu.async_copy(
                    xrv.at[sl], accs[0].at[idx], sem_sc, add=True))
                descs.append(pltpu.async_copy(
                    onesv, accs[1].at[idx], sem_sc, add=True))
                descs.append(pltpu.async_copy(
                    e0v.at[sl], accs[2].at[idx], sem_sc, add=True))
                descs.append(pltpu.async_copy(
                    e1v.at[sl], accs[3].at[idx], sem_sc, add=True))
            pend_sc[ch] = descs

            for d in pend_sc.pop(ch - 1, []):
                d.wait()
            if ch + 1 < _NCHUNK:
                pend_in[ch + 1] = start_in(ch + 1, (ch + 1) % nbuf)

        for chx in sorted(pend_sc):
            for d in pend_sc[chx]:
                d.wait()
        plsc.subcore_barrier()
        sl = pl.ds(sid * _RPS, _RPS)
        for z, acc in enumerate(accs):
            pltpu.sync_copy(acc.at[sl], out_hbm.at[cid, z, sl])

    return k(x_flat, z4, rowl3, col3, e03, e13)


_BB = 8
_F32 = jnp.float32


def _tc_body(xT_ref, alT_ref, accT_ref, suT_ref, sf_ref, h0T_ref, c0T_ref,
             w1T_ref, w2T_ref, b1c_ref, b2c_ref,
             wieT_ref, wheT_ref, benT_ref,
             widT_ref, whdT_ref, bdnT_ref,
             wlinT_ref, blin_ref, out_ref, uproj_s, gxs):
    blk = pl.program_id(0)
    dot = functools.partial(jnp.dot, preferred_element_type=_F32)

    @pl.when(blk == 0)
    def _():
        w2uT = w2T_ref[:, 1 + GH:]
        up = dot(w2uT, suT_ref[...]) + jnp.broadcast_to(b2c_ref[...],
                                                        (GH, N_GRAPH))
        for p in range(T):
            uproj_s[p] = up[:, p * S:(p + 1) * S]

    w2mT = w2T_ref[:, 1:1 + GH]
    aT = dot(w2mT, w1T_ref[...])
    abT = dot(w2mT, b1c_ref[...])
    r6T = jnp.concatenate([w2T_ref[:, 0:1], aT, abT], axis=1)

    for j in range(_BB):
        sx = accT_ref[0, j, 0:1, :] + accT_ref[1, j, 0:1, :]
        cnt = accT_ref[0, j, 1:2, :] + accT_ref[1, j, 1:2, :]
        se0 = accT_ref[0, j, 2:3, :] + accT_ref[1, j, 2:3, :]
        se1 = accT_ref[0, j, 3:4, :] + accT_ref[1, j, 3:4, :]
        rc = 1.0 / jnp.maximum(cnt, 1.0)
        xj = xT_ref[j]
        cts = jnp.concatenate(
            [xj, sx * rc, xj * cnt * rc, se0 * rc, se1 * rc, cnt * rc],
            axis=0)
        for p in range(T):
            gxs[j, p] = (dot(r6T, cts[:, p * S:(p + 1) * S]) + uproj_s[p])

    wieT = wieT_ref[...]
    wheT = wheT_ref[...]
    benb = jnp.broadcast_to(benT_ref[...], (4 * GH, S))

    def sig(x):
        return 0.5 * jnp.tanh(0.5 * x) + 0.5

    def step(h, c, g):
        i = sig(g[0:GH])
        f = sig(g[GH:2 * GH])
        gg = jnp.tanh(g[2 * GH:3 * GH])
        o = sig(g[3 * GH:4 * GH])
        c = f * c + i * gg
        h = o * jnp.tanh(c)
        return h, c

    widT = widT_ref[...]
    whdT = whdT_ref[...]
    wlinT = wlinT_ref[...]
    blv = blin_ref[0, 0]
    wd0T = widT[:, 0:1]
    widrT = widT[:, 1:]
    whd2T = whdT + dot(wd0T, wlinT)
    colmask = (lax.broadcasted_iota(jnp.int32, (1, P), 1) >= 1).astype(_F32)
    dbias = (jnp.broadcast_to(bdnT_ref[...], (4 * GH, P))
             + dot(wd0T * blv, colmask))
    ones1s = jnp.ones((1, S), _F32)

    grp = 4
    for g0 in range(0, _BB, grp):
        js = list(range(g0, g0 + grp))

        def enc_body(p, carry):
            nxt = []
            for q, j in enumerate(js):
                h, c = carry[2 * q], carry[2 * q + 1]
                g = dot(wieT, gxs[j, p]) + dot(wheT, h) + benb
                h, c = step(h, c, g)
                nxt += [h, c]
            return tuple(nxt)

        init = []
        for j in js:
            init += [h0T_ref[j], c0T_ref[j]]
        st = lax.fori_loop(0, T, enc_body, tuple(init))
        hs = {j: st[2 * q] for q, j in enumerate(js)}
        cs = {j: st[2 * q + 1] for q, j in enumerate(js)}

        fpbs = {j: dot(widrT, sf_ref[j]) + dbias for j in js}
        for i in range(P):
            for j in js:
                if i == 0:
                    g = (dot(whdT, hs[j]) + dot(wd0T, alT_ref[j])
                         + dot(fpbs[j][:, 0:1], ones1s))
                else:
                    g = dot(whd2T, hs[j]) + dot(fpbs[j][:, i:i + 1], ones1s)
                hs[j], cs[j] = step(hs[j], cs[j], g)
                out_ref[j, i:i + 1, :] = dot(wlinT, hs[j]) + blv


def _tc_forward(xT, alT, accT, suT, sf, h0T, c0T,
                w1T, w2T, b1c, b2c, wieT, wheT, benT,
                widT, whdT, bdnT, wlinT, blin):
    grid = (B // _BB,)
    full = lambda shape: pl.BlockSpec(shape, lambda b: tuple(0 for _ in shape))
    specs = [
        pl.BlockSpec((_BB, 1, S * T), lambda b: (b, 0, 0)),
        pl.BlockSpec((_BB, 1, S), lambda b: (b, 0, 0)),
        pl.BlockSpec((2, _BB, 4, S * T), lambda b: (0, b, 0, 0)),
        full((43, N_GRAPH)),
        pl.BlockSpec((_BB, 11, P), lambda b: (b, 0, 0)),
        pl.BlockSpec((_BB, GH, S), lambda b: (b, 0, 0)),
        pl.BlockSpec((_BB, GH, S), lambda b: (b, 0, 0)),
        full((GH, 4)), full((GH, 1 + GH + 43)), full((GH, 1)), full((GH, 1)),
        full((4 * GH, GH)), full((4 * GH, GH)), full((4 * GH, 1)),
        full((4 * GH, 12)), full((4 * GH, GH)), full((4 * GH, 1)),
        full((1, GH)), full((1, 1)),
    ]
    return pl.pallas_call(
        _tc_body,
        grid=grid,
        in_specs=specs,
        out_specs=pl.BlockSpec((_BB, P, S), lambda b: (b, 0, 0)),
        out_shape=jax.ShapeDtypeStruct((B, P, S), jnp.float32),
        scratch_shapes=[
            pltpu.VMEM((T, GH, S), jnp.float32),
            pltpu.VMEM((_BB, T, GH, S), jnp.float32),
        ],
        compiler_params=pltpu.CompilerParams(
            dimension_semantics=("arbitrary",),
        ),
    )(xT, alT, accT, suT, sf, h0T, c0T,
      w1T, w2T, b1c, b2c, wieT, wheT, benT,
      widT, whdT, bdnT, wlinT, blin)


def kernel(sta_aqi, sta_conn, sta_w, sta_wea, sta_for, city_u,
           W1, b1, W2, b2, Wi_en, Wh_en, bi_en, bh_en,
           Wi_de, Wh_de, bi_de, bh_de, Wlin, blin, h0, c0):
    conn_t = jnp.transpose(sta_conn, (0, 2, 1))
    conn_g = jnp.tile(conn_t, (T, 1, 1))
    off = (jnp.arange(N_GRAPH, dtype=jnp.int32) * T)[:, None]
    row = (conn_g[:, 0, :] + off).reshape(-1)
    col = (conn_g[:, 1, :] + off).reshape(-1)
    widx = jnp.arange(N_EDGE, dtype=jnp.int32) // _EPW
    rowl = row - widx * _XBASE
    ew = sta_w.reshape(-1, 2)
    x_flat = sta_aqi.reshape(-1)

    rowl3 = rowl.reshape(N_EDGE // _CH, _CH)
    col3 = col.reshape(N_EDGE // 128, 128).reshape(N_EDGE // _CH, 8, 128)
    e03 = ew[:, 0].reshape(N_EDGE // _CH, _CH)
    e13 = ew[:, 1].reshape(N_EDGE // _CH, _CH)

    z4 = jnp.zeros((_RPS,), jnp.float32)
    acc = _sc_segment_acc(x_flat, z4, rowl3, col3, e03, e13)
    accT = jnp.transpose(acc.reshape(2, 4, B, S * T), (0, 2, 1, 3))

    xT = x_flat.reshape(B, 1, S * T)
    alT = sta_aqi[:, :, -1, 0].reshape(B, 1, S)
    suT = jnp.transpose(
        jnp.concatenate([city_u, sta_wea], axis=-1).reshape(N_GRAPH, 43))
    sfT = jnp.transpose(sta_for, (0, 2, 1))
    h0T = jnp.transpose(h0.reshape(B, S, GH), (0, 2, 1))
    c0T = jnp.transpose(c0.reshape(B, S, GH), (0, 2, 1))

    out = _tc_forward(
        xT, alT, accT, suT, sfT, h0T, c0T,
        jnp.transpose(W1), jnp.transpose(W2),
        b1.reshape(GH, 1), b2.reshape(GH, 1),
        jnp.transpose(Wi_en), jnp.transpose(Wh_en),
        (bi_en + bh_en).reshape(4 * GH, 1),
        jnp.transpose(Wi_de), jnp.transpose(Wh_de),
        (bi_de + bh_de).reshape(4 * GH, 1),
        jnp.transpose(Wlin), blin.reshape(1, 1))
    return jnp.transpose(out, (0, 2, 1))


# speedup vs baseline: 1.0638x; 1.0305x over previous
import functools

import jax
import jax.numpy as jnp
from jax import lax
from jax.experimental import pallas as pl
from jax.experimental.pallas import tpu as pltpu
from jax.experimental.pallas import tpu_sc as plsc

B, S, T, P, E, GH = 64, 64, 24, 24, 512, 32
N_NODE = B * S * T
N_EDGE = T * B * E
N_GRAPH = T * B

_NC, _NS = 2, 16
_NW = _NC * _NS
_EPW = N_EDGE // _NW
_GPW = N_GRAPH // _NW
_XBASE = _GPW * T
_XSL = 1200
_CH = 1024
_NCHUNK = _EPW // _CH
_RPS = N_NODE // _NS


def _sc_segment_acc(x_flat, z4, rowl3, col3, e03, e13):
    mesh = plsc.VectorSubcoreMesh(core_axis_name="c", subcore_axis_name="s")
    nbuf = 2

    @functools.partial(
        pl.kernel,
        mesh=mesh,
        out_type=jax.ShapeDtypeStruct((_NC, 4, N_NODE), jnp.float32),
        scratch_types=[
            pltpu.VMEM((_XSL,), jnp.float32),
            [pltpu.VMEM((_CH,), jnp.int32) for _ in range(nbuf)],
            [pltpu.VMEM((8, 128), jnp.int32) for _ in range(nbuf)],
            [pltpu.VMEM((_CH,), jnp.float32) for _ in range(nbuf)],
            [pltpu.VMEM((_CH,), jnp.float32) for _ in range(nbuf)],
            [pltpu.VMEM((_CH,), jnp.float32) for _ in range(nbuf)],
            pltpu.VMEM((128,), jnp.float32),
            [pltpu.VMEM_SHARED((N_NODE,), jnp.float32) for _ in range(4)],
            pltpu.SemaphoreType.DMA,
            pltpu.SemaphoreType.DMA,
        ],
        compiler_params=pltpu.CompilerParams(needs_layout_passes=False),
    )
    def k(x_hbm, z_hbm, rowl_hbm, col_hbm, e0_hbm, e1_hbm, out_hbm,
          xv, rls, cols, e0s, e1s, xrs, onesv, accs, sem_in, sem_sc):
        cid = lax.axis_index("c")
        sid = lax.axis_index("s")
        wid = sid * _NC + cid

        for z, acc in enumerate(accs):
            pltpu.sync_copy(z_hbm, acc.at[pl.ds(sid * _RPS, _RPS)])

        def fill1(i, _):
            onesv[pl.ds(i * 16, 16)] = jnp.ones((16,), jnp.float32)
            return 0
        lax.fori_loop(0, 8, fill1, 0)
        plsc.subcore_barrier()

        pltpu.sync_copy(x_hbm.at[pl.ds(wid * _XBASE, _XSL)], xv)

        def start_in(ch, kb):
            cg = wid * _NCHUNK + ch
            return [
                pltpu.async_copy(rowl_hbm.at[cg], rls[kb], sem_in),
                pltpu.async_copy(col_hbm.at[cg], cols[kb], sem_in),
                pltpu.async_copy(e0_hbm.at[cg], e0s[kb], sem_in),
                pltpu.async_copy(e1_hbm.at[cg], e1s[kb], sem_in),
            ]

        pend_in = {0: start_in(0, 0)}
        pend_sc = {}
        for ch in range(_NCHUNK):
            kb = ch % nbuf
            for d in pend_in.pop(ch):
                d.wait()
            rl, e0v, e1v, xrv = rls[kb], e0s[kb], e1s[kb], xrs[kb]

            def build(j, _):
                base = j * 16
                r = rl[pl.ds(base, 16)]
                xrv[pl.ds(base, 16)] = plsc.load_gather(xv, [r])
                return 0
            lax.fori_loop(0, _CH // 16, build, 0)

            descs = []
            for r8 in range(8):
                idx = cols[kb].at[r8]
                sl = pl.ds(r8 * 128, 128)
                descs.append(pltpu.async_copy(
                    xrv.at[sl], accs[0].at[idx], sem_sc, add=True))
                descs.append(pltpu.async_copy(
                    onesv, accs[1].at[idx], sem_sc, add=True))
                descs.append(pltpu.async_copy(
                    e0v.at[sl], accs[2].at[idx], sem_sc, add=True))
                descs.append(pltpu.async_copy(
                    e1v.at[sl], accs[3].at[idx], sem_sc, add=True))
            pend_sc[ch] = descs

            for d in pend_sc.pop(ch - 1, []):
                d.wait()
            if ch + 1 < _NCHUNK:
                pend_in[ch + 1] = start_in(ch + 1, (ch + 1) % nbuf)

        for chx in sorted(pend_sc):
            for d in pend_sc[chx]:
                d.wait()
        plsc.subcore_barrier()
        sl = pl.ds(sid * _RPS, _RPS)
        for z, acc in enumerate(accs):
            pltpu.sync_copy(acc.at[sl], out_hbm.at[cid, z, sl])

    return k(x_flat, z4, rowl3, col3, e03, e13)


_BB = 8
_F32 = jnp.float32


def _tc_body(xT_ref, alT_ref, accT_ref, suT_ref, sf_ref, h0T_ref, c0T_ref,
             w1T_ref, w2T_ref, b1c_ref, b2c_ref,
             wieT_ref, wheT_ref, benT_ref,
             widT_ref, whdT_ref, bdnT_ref,
             wlinT_ref, blin_ref, out_ref, uproj_s, gxs):
    blk = pl.program_id(0)
    dot = functools.partial(jnp.dot, preferred_element_type=_F32)

    @pl.when(blk == 0)
    def _():
        w2uT = w2T_ref[:, 1 + GH:]
        up = dot(w2uT, suT_ref[...]) + jnp.broadcast_to(b2c_ref[...],
                                                        (GH, N_GRAPH))
        for p in range(T):
            uproj_s[p] = up[:, p * S:(p + 1) * S]

    w2mT = w2T_ref[:, 1:1 + GH]
    aT = dot(w2mT, w1T_ref[...])
    abT = dot(w2mT, b1c_ref[...])
    r6T = jnp.concatenate([w2T_ref[:, 0:1], aT, abT], axis=1)

    for j in range(_BB):
        sx = accT_ref[0, j, 0:1, :] + accT_ref[1, j, 0:1, :]
        cnt = accT_ref[0, j, 1:2, :] + accT_ref[1, j, 1:2, :]
        se0 = accT_ref[0, j, 2:3, :] + accT_ref[1, j, 2:3, :]
        se1 = accT_ref[0, j, 3:4, :] + accT_ref[1, j, 3:4, :]
        rc = 1.0 / jnp.maximum(cnt, 1.0)
        xj = xT_ref[j]
        cts = jnp.concatenate(
            [xj, sx * rc, xj * cnt * rc, se0 * rc, se1 * rc, cnt * rc],
            axis=0)
        for p in range(T):
            gxs[j, p] = (dot(r6T, cts[:, p * S:(p + 1) * S]) + uproj_s[p])

    wieT = wieT_ref[...]
    wheT = wheT_ref[...]
    benb = jnp.broadcast_to(benT_ref[...], (4 * GH, S))

    def step(h, c, g):
        i = jax.nn.sigmoid(g[0:GH])
        f = jax.nn.sigmoid(g[GH:2 * GH])
        gg = jnp.tanh(g[2 * GH:3 * GH])
        o = jax.nn.sigmoid(g[3 * GH:4 * GH])
        c = f * c + i * gg
        h = o * jnp.tanh(c)
        return h, c

    widT = widT_ref[...]
    whdT = whdT_ref[...]
    wlinT = wlinT_ref[...]
    blv = blin_ref[0, 0]
    wd0T = widT[:, 0:1]
    widrT = widT[:, 1:]
    whd2T = whdT + dot(wd0T, wlinT)
    colmask = (lax.broadcasted_iota(jnp.int32, (1, P), 1) >= 1).astype(_F32)
    dbias = (jnp.broadcast_to(bdnT_ref[...], (4 * GH, P))
             + dot(wd0T * blv, colmask))
    ones1s = jnp.ones((1, S), _F32)

    def enc_body(p, carry):
        hsA, csA = carry
        nh, nc = [], []
        for j in range(_BB):
            g = dot(wieT, gxs[j, p]) + dot(wheT, hsA[j]) + benb
            h, c = step(hsA[j], csA[j], g)
            nh.append(h)
            nc.append(c)
        return jnp.stack(nh), jnp.stack(nc)

    hsA, csA = lax.fori_loop(0, T, enc_body, (h0T_ref[...], c0T_ref[...]))
    hs = [hsA[j] for j in range(_BB)]
    cs = [csA[j] for j in range(_BB)]

    fpbs = [dot(widrT, sf_ref[j]) + dbias for j in range(_BB)]
    for i in range(P):
        for j in range(_BB):
            if i == 0:
                g = (dot(whdT, hs[j]) + dot(wd0T, alT_ref[j])
                     + dot(fpbs[j][:, 0:1], ones1s))
            else:
                g = dot(whd2T, hs[j]) + dot(fpbs[j][:, i:i + 1], ones1s)
            hs[j], cs[j] = step(hs[j], cs[j], g)
            out_ref[j, i:i + 1, :] = dot(wlinT, hs[j]) + blv


def _tc_forward(xT, alT, accT, suT, sf, h0T, c0T,
                w1T, w2T, b1c, b2c, wieT, wheT, benT,
                widT, whdT, bdnT, wlinT, blin):
    grid = (B // _BB,)
    full = lambda shape: pl.BlockSpec(shape, lambda b: tuple(0 for _ in shape))
    specs = [
        pl.BlockSpec((_BB, 1, S * T), lambda b: (b, 0, 0)),
        pl.BlockSpec((_BB, 1, S), lambda b: (b, 0, 0)),
        pl.BlockSpec((2, _BB, 4, S * T), lambda b: (0, b, 0, 0)),
        full((43, N_GRAPH)),
        pl.BlockSpec((_BB, 11, P), lambda b: (b, 0, 0)),
        pl.BlockSpec((_BB, GH, S), lambda b: (b, 0, 0)),
        pl.BlockSpec((_BB, GH, S), lambda b: (b, 0, 0)),
        full((GH, 4)), full((GH, 1 + GH + 43)), full((GH, 1)), full((GH, 1)),
        full((4 * GH, GH)), full((4 * GH, GH)), full((4 * GH, 1)),
        full((4 * GH, 12)), full((4 * GH, GH)), full((4 * GH, 1)),
        full((1, GH)), full((1, 1)),
    ]
    return pl.pallas_call(
        _tc_body,
        grid=grid,
        in_specs=specs,
        out_specs=pl.BlockSpec((_BB, P, S), lambda b: (b, 0, 0)),
        out_shape=jax.ShapeDtypeStruct((B, P, S), jnp.float32),
        scratch_shapes=[
            pltpu.VMEM((T, GH, S), jnp.float32),
            pltpu.VMEM((_BB, T, GH, S), jnp.float32),
        ],
        compiler_params=pltpu.CompilerParams(
            dimension_semantics=("arbitrary",),
        ),
    )(xT, alT, accT, suT, sf, h0T, c0T,
      w1T, w2T, b1c, b2c, wieT, wheT, benT,
      widT, whdT, bdnT, wlinT, blin)


def kernel(sta_aqi, sta_conn, sta_w, sta_wea, sta_for, city_u,
           W1, b1, W2, b2, Wi_en, Wh_en, bi_en, bh_en,
           Wi_de, Wh_de, bi_de, bh_de, Wlin, blin, h0, c0):
    conn_t = jnp.transpose(sta_conn, (0, 2, 1))
    conn_g = jnp.tile(conn_t, (T, 1, 1))
    off = (jnp.arange(N_GRAPH, dtype=jnp.int32) * T)[:, None]
    row = (conn_g[:, 0, :] + off).reshape(-1)
    col = (conn_g[:, 1, :] + off).reshape(-1)
    widx = jnp.arange(N_EDGE, dtype=jnp.int32) // _EPW
    rowl = row - widx * _XBASE
    ew = sta_w.reshape(-1, 2)
    x_flat = sta_aqi.reshape(-1)

    rowl3 = rowl.reshape(N_EDGE // _CH, _CH)
    col3 = col.reshape(N_EDGE // 128, 128).reshape(N_EDGE // _CH, 8, 128)
    e03 = ew[:, 0].reshape(N_EDGE // _CH, _CH)
    e13 = ew[:, 1].reshape(N_EDGE // _CH, _CH)

    z4 = jnp.zeros((_RPS,), jnp.float32)
    acc = _sc_segment_acc(x_flat, z4, rowl3, col3, e03, e13)
    accT = jnp.transpose(acc.reshape(2, 4, B, S * T), (0, 2, 1, 3))

    xT = x_flat.reshape(B, 1, S * T)
    alT = sta_aqi[:, :, -1, 0].reshape(B, 1, S)
    suT = jnp.transpose(
        jnp.concatenate([city_u, sta_wea], axis=-1).reshape(N_GRAPH, 43))
    sfT = jnp.transpose(sta_for, (0, 2, 1))
    h0T = jnp.transpose(h0.reshape(B, S, GH), (0, 2, 1))
    c0T = jnp.transpose(c0.reshape(B, S, GH), (0, 2, 1))

    out = _tc_forward(
        xT, alT, accT, suT, sfT, h0T, c0T,
        jnp.transpose(W1), jnp.transpose(W2),
        b1.reshape(GH, 1), b2.reshape(GH, 1),
        jnp.transpose(Wi_en), jnp.transpose(Wh_en),
        (bi_en + bh_en).reshape(4 * GH, 1),
        jnp.transpose(Wi_de), jnp.transpose(Wh_de),
        (bi_de + bh_de).reshape(4 * GH, 1),
        jnp.transpose(Wlin), blin.reshape(1, 1))
    return jnp.transpose(out, (0, 2, 1))


# speedup vs baseline: 1.9377x; 1.8216x over previous
import functools

import jax
import jax.numpy as jnp
from jax import lax
from jax.experimental import pallas as pl
from jax.experimental.pallas import tpu as pltpu
from jax.experimental.pallas import tpu_sc as plsc

B, S, T, P, E, GH = 64, 64, 24, 24, 512, 32
N_NODE = B * S * T
N_EDGE = T * B * E
N_GRAPH = T * B

_NC, _NS = 2, 16
_NW = _NC * _NS
_EPW = N_EDGE // _NW
_GPW = N_GRAPH // _NW
_XBASE = _GPW * T
_XSL = 1200
_CH = 1024
_NCHUNK = _EPW // _CH
_RPS = N_NODE // _NS


def _sc_segment_acc(x_flat, z4, rowl3, col3, e03, e13):
    mesh = plsc.VectorSubcoreMesh(core_axis_name="c", subcore_axis_name="s")
    nbuf = 2

    @functools.partial(
        pl.kernel,
        mesh=mesh,
        out_type=jax.ShapeDtypeStruct((_NC, 4, N_NODE), jnp.float32),
        scratch_types=[
            pltpu.VMEM((_XSL,), jnp.float32),
            [pltpu.VMEM((_CH,), jnp.int32) for _ in range(nbuf)],
            [pltpu.VMEM((8, 128), jnp.int32) for _ in range(nbuf)],
            [pltpu.VMEM((_CH,), jnp.float32) for _ in range(nbuf)],
            [pltpu.VMEM((_CH,), jnp.float32) for _ in range(nbuf)],
            [pltpu.VMEM((_CH,), jnp.float32) for _ in range(nbuf)],
            pltpu.VMEM((128,), jnp.float32),
            [pltpu.VMEM_SHARED((N_NODE,), jnp.float32) for _ in range(4)],
            pltpu.SemaphoreType.DMA,
            pltpu.SemaphoreType.DMA,
        ],
        compiler_params=pltpu.CompilerParams(needs_layout_passes=False),
    )
    def k(x_hbm, z_hbm, rowl_hbm, col_hbm, e0_hbm, e1_hbm, out_hbm,
          xv, rls, cols, e0s, e1s, xrs, onesv, accs, sem_in, sem_sc):
        cid = lax.axis_index("c")
        sid = lax.axis_index("s")
        wid = sid * _NC + cid

        for z, acc in enumerate(accs):
            pltpu.sync_copy(z_hbm, acc.at[pl.ds(sid * _RPS, _RPS)])

        def fill1(i, _):
            onesv[pl.ds(i * 16, 16)] = jnp.ones((16,), jnp.float32)
            return 0
        lax.fori_loop(0, 8, fill1, 0)
        plsc.subcore_barrier()

        pltpu.sync_copy(x_hbm.at[pl.ds(wid * _XBASE, _XSL)], xv)

        def start_in(ch, kb):
            cg = wid * _NCHUNK + ch
            return [
                pltpu.async_copy(rowl_hbm.at[cg], rls[kb], sem_in),
                pltpu.async_copy(col_hbm.at[cg], cols[kb], sem_in),
                pltpu.async_copy(e0_hbm.at[cg], e0s[kb], sem_in),
                pltpu.async_copy(e1_hbm.at[cg], e1s[kb], sem_in),
            ]

        pend_in = {0: start_in(0, 0)}
        pend_sc = {}
        for ch in range(_NCHUNK):
            kb = ch % nbuf
            for d in pend_in.pop(ch):
                d.wait()
            rl, e0v, e1v, xrv = rls[kb], e0s[kb], e1s[kb], xrs[kb]

            def build(j, _):
                base = j * 16
                r = rl[pl.ds(base, 16)]
                xrv[pl.ds(base, 16)] = plsc.load_gather(xv, [r])
                return 0
            lax.fori_loop(0, _CH // 16, build, 0)

            descs = []
            for r8 in range(8):
                idx = cols[kb].at[r8]
                sl = pl.ds(r8 * 128, 128)
                descs.append(pltpu.async_copy(
                    xrv.at[sl], accs[0].at[idx], sem_sc, add=True))
                descs.append(pltpu.async_copy(
                    onesv, accs[1].at[idx], sem_sc, add=True))
                descs.append(pltpu.async_copy(
                    e0v.at[sl], accs[2].at[idx], sem_sc, add=True))
                descs.append(pltpu.async_copy(
                    e1v.at[sl], accs[3].at[idx], sem_sc, add=True))
            pend_sc[ch] = descs

            for d in pend_sc.pop(ch - 1, []):
                d.wait()
            if ch + 1 < _NCHUNK:
                pend_in[ch + 1] = start_in(ch + 1, (ch + 1) % nbuf)

        for chx in sorted(pend_sc):
            for d in pend_sc[chx]:
                d.wait()
        plsc.subcore_barrier()
        sl = pl.ds(sid * _RPS, _RPS)
        for z, acc in enumerate(accs):
            pltpu.sync_copy(acc.at[sl], out_hbm.at[cid, z, sl])

    return k(x_flat, z4, rowl3, col3, e03, e13)


_BB = 8
_F32 = jnp.float32


def _tc_body(xT_ref, alT_ref, accT_ref, suT_ref, sf_ref, h0T_ref, c0T_ref,
             w1T_ref, w2T_ref, b1c_ref, b2c_ref,
             wieT_ref, wheT_ref, benT_ref,
             widT_ref, whdT_ref, bdnT_ref,
             wlinT_ref, blin_ref, out_ref, uproj_s, gxs, wblk_s, hsave):
    blk = pl.program_id(0)
    dot = functools.partial(jnp.dot, preferred_element_type=_F32)

    @pl.when(blk == 0)
    def _():
        w2uT = w2T_ref[:, 1 + GH:]
        up = dot(w2uT, suT_ref[...]) + jnp.broadcast_to(b2c_ref[...],
                                                        (GH, N_GRAPH))
        for p in range(T):
            uproj_s[p] = up[:, p * S:(p + 1) * S]

    w2mT = w2T_ref[:, 1:1 + GH]
    aT = dot(w2mT, w1T_ref[...])
    abT = dot(w2mT, b1c_ref[...])
    r6T = jnp.concatenate([w2T_ref[:, 0:1], aT, abT], axis=1)

    for j in range(_BB):
        sx = accT_ref[0, j, 0:1, :] + accT_ref[1, j, 0:1, :]
        cnt = accT_ref[0, j, 1:2, :] + accT_ref[1, j, 1:2, :]
        se0 = accT_ref[0, j, 2:3, :] + accT_ref[1, j, 2:3, :]
        se1 = accT_ref[0, j, 3:4, :] + accT_ref[1, j, 3:4, :]
        rc = 1.0 / jnp.maximum(cnt, 1.0)
        xj = xT_ref[j]
        cts = jnp.concatenate(
            [xj, sx * rc, xj * cnt * rc, se0 * rc, se1 * rc, cnt * rc],
            axis=0)
        for p in range(T):
            gxs[j, p] = (dot(r6T, cts[:, p * S:(p + 1) * S]) + uproj_s[p])

    wcat = jnp.concatenate([wieT_ref[...], wheT_ref[...]], axis=1)
    benb = jnp.broadcast_to(benT_ref[...], (4 * GH, S))

    def step(h, c, g):
        i = jax.nn.sigmoid(g[0:GH])
        f = jax.nn.sigmoid(g[GH:2 * GH])
        gg = jnp.tanh(g[2 * GH:3 * GH])
        o = jax.nn.sigmoid(g[3 * GH:4 * GH])
        c = f * c + i * gg
        h = o * jnp.tanh(c)
        return h, c

    widT = widT_ref[...]
    whdT = whdT_ref[...]
    wlinT = wlinT_ref[...]
    blv = blin_ref[0, 0]
    wd0T = widT[:, 0:1]
    widrT = widT[:, 1:]
    whd2T = whdT + dot(wd0T, wlinT)
    colmask = (lax.broadcasted_iota(jnp.int32, (1, P), 1) >= 1).astype(_F32)
    dbias = (jnp.broadcast_to(bdnT_ref[...], (4 * GH, P))
             + dot(wd0T * blv, colmask))
    ones1s = jnp.ones((1, S), _F32)

    def enc_body(p, carry):
        hsA, csA = carry
        nh, nc = [], []
        for j in range(_BB):
            xin = jnp.concatenate([gxs[j, p], hsA[j]], axis=0)
            g = dot(wcat, xin) + benb
            h, c = step(hsA[j], csA[j], g)
            nh.append(h)
            nc.append(c)
        return jnp.stack(nh), jnp.stack(nc)

    hsA, csA = lax.fori_loop(0, T, enc_body, (h0T_ref[...], c0T_ref[...]))
    hs = [hsA[j] for j in range(_BB)]
    cs = [csA[j] for j in range(_BB)]

    @pl.when(blk == 0)
    def _():
        wblk_s[...] = jnp.zeros((P, P * GH), _F32)
        for i in range(P):
            wblk_s[i:i + 1, i * GH:(i + 1) * GH] = wlinT

    fpbs = [dot(widrT, sf_ref[j]) + dbias for j in range(_BB)]
    for i in range(P):
        for j in range(_BB):
            if i == 0:
                g = (dot(whdT, hs[j]) + dot(wd0T, alT_ref[j])
                     + dot(fpbs[j][:, 0:1], ones1s))
            else:
                g = dot(whd2T, hs[j]) + dot(fpbs[j][:, i:i + 1], ones1s)
            hs[j], cs[j] = step(hs[j], cs[j], g)
            hsave[j, i] = hs[j]
    wblk = wblk_s[...]
    for j in range(_BB):
        hflat = hsave[j].reshape(P * GH, S)
        out_ref[j] = dot(wblk, hflat) + blv


def _tc_forward(xT, alT, accT, suT, sf, h0T, c0T,
                w1T, w2T, b1c, b2c, wieT, wheT, benT,
                widT, whdT, bdnT, wlinT, blin):
    grid = (B // _BB,)
    full = lambda shape: pl.BlockSpec(shape, lambda b: tuple(0 for _ in shape))
    specs = [
        pl.BlockSpec((_BB, 1, S * T), lambda b: (b, 0, 0)),
        pl.BlockSpec((_BB, 1, S), lambda b: (b, 0, 0)),
        pl.BlockSpec((2, _BB, 4, S * T), lambda b: (0, b, 0, 0)),
        full((43, N_GRAPH)),
        pl.BlockSpec((_BB, 11, P), lambda b: (b, 0, 0)),
        pl.BlockSpec((_BB, GH, S), lambda b: (b, 0, 0)),
        pl.BlockSpec((_BB, GH, S), lambda b: (b, 0, 0)),
        full((GH, 4)), full((GH, 1 + GH + 43)), full((GH, 1)), full((GH, 1)),
        full((4 * GH, GH)), full((4 * GH, GH)), full((4 * GH, 1)),
        full((4 * GH, 12)), full((4 * GH, GH)), full((4 * GH, 1)),
        full((1, GH)), full((1, 1)),
    ]
    return pl.pallas_call(
        _tc_body,
        grid=grid,
        in_specs=specs,
        out_specs=pl.BlockSpec((_BB, P, S), lambda b: (b, 0, 0)),
        out_shape=jax.ShapeDtypeStruct((B, P, S), jnp.float32),
        scratch_shapes=[
            pltpu.VMEM((T, GH, S), jnp.float32),
            pltpu.VMEM((_BB, T, GH, S), jnp.float32),
            pltpu.VMEM((P, P * GH), jnp.float32),
            pltpu.VMEM((_BB, P, GH, S), jnp.float32),
        ],
        compiler_params=pltpu.CompilerParams(
            dimension_semantics=("arbitrary",),
        ),
    )(xT, alT, accT, suT, sf, h0T, c0T,
      w1T, w2T, b1c, b2c, wieT, wheT, benT,
      widT, whdT, bdnT, wlinT, blin)


def kernel(sta_aqi, sta_conn, sta_w, sta_wea, sta_for, city_u,
           W1, b1, W2, b2, Wi_en, Wh_en, bi_en, bh_en,
           Wi_de, Wh_de, bi_de, bh_de, Wlin, blin, h0, c0):
    conn_t = jnp.transpose(sta_conn, (0, 2, 1))
    conn_g = jnp.tile(conn_t, (T, 1, 1))
    off = (jnp.arange(N_GRAPH, dtype=jnp.int32) * T)[:, None]
    row = (conn_g[:, 0, :] + off).reshape(-1)
    col = (conn_g[:, 1, :] + off).reshape(-1)
    widx = jnp.arange(N_EDGE, dtype=jnp.int32) // _EPW
    rowl = row - widx * _XBASE
    ew = sta_w.reshape(-1, 2)
    x_flat = sta_aqi.reshape(-1)

    rowl3 = rowl.reshape(N_EDGE // _CH, _CH)
    col3 = col.reshape(N_EDGE // 128, 128).reshape(N_EDGE // _CH, 8, 128)
    e03 = ew[:, 0].reshape(N_EDGE // _CH, _CH)
    e13 = ew[:, 1].reshape(N_EDGE // _CH, _CH)

    z4 = jnp.zeros((_RPS,), jnp.float32)
    acc = _sc_segment_acc(x_flat, z4, rowl3, col3, e03, e13)
    accT = jnp.transpose(acc.reshape(2, 4, B, S * T), (0, 2, 1, 3))

    xT = x_flat.reshape(B, 1, S * T)
    alT = sta_aqi[:, :, -1, 0].reshape(B, 1, S)
    suT = jnp.transpose(
        jnp.concatenate([city_u, sta_wea], axis=-1).reshape(N_GRAPH, 43))
    sfT = jnp.transpose(sta_for, (0, 2, 1))
    h0T = jnp.transpose(h0.reshape(B, S, GH), (0, 2, 1))
    c0T = jnp.transpose(c0.reshape(B, S, GH), (0, 2, 1))

    out = _tc_forward(
        xT, alT, accT, suT, sfT, h0T, c0T,
        jnp.transpose(W1), jnp.transpose(W2),
        b1.reshape(GH, 1), b2.reshape(GH, 1),
        jnp.transpose(Wi_en), jnp.transpose(Wh_en),
        (bi_en + bh_en).reshape(4 * GH, 1),
        jnp.transpose(Wi_de), jnp.transpose(Wh_de),
        (bi_de + bh_de).reshape(4 * GH, 1),
        jnp.transpose(Wlin), blin.reshape(1, 1))
    return jnp.transpose(out, (0, 2, 1))
